# baseline (speedup 1.0000x reference)
import functools

import jax
import jax.numpy as jnp
from jax import lax
from jax.experimental import pallas as pl
from jax.experimental.pallas import tpu as pltpu
from jax.experimental.pallas import tpu_sc as plsc

D_MODEL = 32
NUM_CORES = 2
NUM_SUBCORES = 16
NW = NUM_CORES * NUM_SUBCORES
GROUP = 128
K = 13


@functools.partial(jax.jit, static_argnames=("nchunk",))
def _gather_rows(idx, table, nchunk):
    G = nchunk * K
    mesh = plsc.VectorSubcoreMesh(core_axis_name="c", subcore_axis_name="s")

    @functools.partial(
        pl.kernel,
        out_type=jax.ShapeDtypeStruct((NW, G, GROUP, D_MODEL), jnp.float32),
        mesh=mesh,
        scratch_types=[
            pltpu.VMEM((G, GROUP), jnp.int32),
            pltpu.VMEM((2, K, GROUP, D_MODEL), jnp.float32),
            pltpu.SemaphoreType.DMA,
            pltpu.SemaphoreType.DMA,
            pltpu.SemaphoreType.DMA,
        ],
        compiler_params=pltpu.CompilerParams(use_tc_tiling_on_sc=False),
    )
    def k(table_hbm, idx_hbm, out_hbm, idx_v, rows_v, gsem0, gsem1, ssem):
        wid = lax.axis_index("s") * NUM_CORES + lax.axis_index("c")
        pltpu.sync_copy(idx_hbm.at[wid], idx_v)

        def fire(c, region, sem):
            for j in range(K):
                pltpu.async_copy(
                    table_hbm.at[idx_v.at[c * K + j]],
                    rows_v.at[region, j],
                    sem,
                )

        fire(0, 0, gsem0)

        def step(c, carry):
            r = lax.rem(c, 2)
            cur_sem_is0 = lax.rem(c, 2) == 0

            @pl.when(c >= 1)
            def _():
                pltpu.make_async_copy(
                    rows_v.at[0], out_hbm.at[wid, pl.ds(0, K)], ssem
                ).wait()

            @pl.when(c + 1 < nchunk)
            def _():
                @pl.when(cur_sem_is0)
                def _():
                    fire(c + 1, 1 - r, gsem1)

                @pl.when(jnp.logical_not(cur_sem_is0))
                def _():
                    fire(c + 1, 1 - r, gsem0)

            @pl.when(cur_sem_is0)
            def _():
                pltpu.make_async_copy(
                    out_hbm.at[wid, pl.ds(0, K)], rows_v.at[r], gsem0
                ).wait()

            @pl.when(jnp.logical_not(cur_sem_is0))
            def _():
                pltpu.make_async_copy(
                    out_hbm.at[wid, pl.ds(0, K)], rows_v.at[r], gsem1
                ).wait()

            pltpu.async_copy(rows_v.at[r], out_hbm.at[wid, pl.ds(c * K, K)], ssem)
            return carry

        lax.fori_loop(0, nchunk, step, 0)
        pltpu.make_async_copy(
            rows_v.at[0], out_hbm.at[wid, pl.ds(0, K)], ssem
        ).wait()

    return k(table, idx)


def kernel(x, table):
    B_total = x.shape[0] * x.shape[1]
    chunk = NW * GROUP * K
    B_pad = ((B_total + chunk - 1) // chunk) * chunk
    nchunk = B_pad // chunk
    G = nchunk * K
    xf = x.reshape(-1)
    if B_pad != B_total:
        xf = jnp.concatenate(
            [xf, jnp.zeros((B_pad - B_total,), dtype=xf.dtype)]
        )
    idx = xf.reshape(NW, G, GROUP)
    rows = _gather_rows(idx, table, nchunk)
    rows = rows.reshape(B_pad, D_MODEL)[:B_total]
    return rows.reshape(x.shape[0], x.shape[1], D_MODEL)
